# baseline (speedup 1.0000x reference)
import functools

import jax
import jax.numpy as jnp
from jax import lax
from jax.experimental import pallas as pl
from jax.experimental.pallas import tpu as pltpu
from jax.experimental.pallas import tpu_sc as plsc

VOC = 1000000
D = 64
B = 16384
SEQ = 5
H = 128

NC = 2
NS = 16
NW = NC * NS
N_IDX = B * SEQ
ROWS_PER_W = N_IDX // NW
CHUNK = 128
NCHUNK = ROWS_PER_W // CHUNK


def _sc_gather(idx_3d, E):
    mesh = plsc.VectorSubcoreMesh(core_axis_name="c", subcore_axis_name="s")

    @functools.partial(
        pl.kernel,
        out_type=jax.ShapeDtypeStruct((N_IDX, D), jnp.float32),
        mesh=mesh,
        scratch_types=[
            pltpu.VMEM((NCHUNK, CHUNK), jnp.int32),
            pltpu.VMEM((2, CHUNK, D), jnp.float32),
            pltpu.SemaphoreType.DMA((2,)),
            pltpu.SemaphoreType.DMA((2,)),
        ],
        compiler_params=pltpu.CompilerParams(use_tc_tiling_on_sc=False),
    )
    def gather_kernel(idx_hbm, table_hbm, out_hbm, idx_v, rows_v, gsem, osem):
        wid = lax.axis_index("s") * NC + lax.axis_index("c")
        base = wid * ROWS_PER_W
        pltpu.sync_copy(idx_hbm.at[wid], idx_v)

        def start_gather(j, b):
            return pltpu.async_copy(
                table_hbm.at[idx_v.at[j]], rows_v.at[b], gsem.at[b]
            )

        def start_out(j, b):
            return pltpu.async_copy(
                rows_v.at[b], out_hbm.at[pl.ds(base + j * CHUNK, CHUNK)],
                osem.at[b],
            )

        copies = {}
        copies[("g", 0)] = start_gather(0, 0)
        for j in range(NCHUNK):
            b = j % 2
            if j + 1 < NCHUNK:
                b2 = (j + 1) % 2
                if j >= 1:
                    copies[("o", j - 1)].wait()
                copies[("g", j + 1)] = start_gather(j + 1, b2)
            copies[("g", j)].wait()
            copies[("o", j)] = start_out(j, b)
        copies[("o", NCHUNK - 2)].wait()
        copies[("o", NCHUNK - 1)].wait()

    return gather_kernel(idx_3d, E)


def _mlp_body(emb_ref, w1s_ref, b1s_ref, w2s_ref, b2s_ref,
              w1n_ref, b1n_ref, w2n_ref, b2n_ref,
              synt0_ref, sent0_ref, synt1_ref, sent1_ref, synt2_ref, sent2_ref):
    g01 = emb_ref[:, :2 * D]
    p_s = jnp.dot(g01, w1s_ref[:2 * D, :], preferred_element_type=jnp.float32)
    p_n = jnp.dot(g01, w1n_ref[:2 * D, :], preferred_element_type=jnp.float32)
    w1s_c = w1s_ref[2 * D:, :]
    w1n_c = w1n_ref[2 * D:, :]
    synt_outs = (synt0_ref, synt1_ref, synt2_ref)
    sent_outs = (sent0_ref, sent1_ref, sent2_ref)
    for k in range(3):
        ek = emb_ref[:, (2 + k) * D:(3 + k) * D]
        hs = jnp.clip(
            p_s + jnp.dot(ek, w1s_c, preferred_element_type=jnp.float32)
            + b1s_ref[0, :], -1.0, 1.0)
        synt = jnp.sum(hs * w2s_ref[0:1, :], axis=1, keepdims=True) + b2s_ref[0, 0]
        synt_outs[k][...] = synt
        hn = jnp.clip(
            p_n + jnp.dot(ek, w1n_c, preferred_element_type=jnp.float32)
            + b1n_ref[0, :], -1.0, 1.0)
        l0 = jnp.sum(hn * w2n_ref[0:1, :], axis=1, keepdims=True) + b2n_ref[0, 0]
        l1 = jnp.sum(hn * w2n_ref[1:2, :], axis=1, keepdims=True) + b2n_ref[0, 1]
        m = jnp.maximum(l0, l1)
        e0 = jnp.exp(l0 - m)
        e1 = jnp.exp(l1 - m)
        inv = 1.0 / (e0 + e1)
        sent_outs[k][...] = jnp.concatenate([e0 * inv, e1 * inv], axis=1)


def _tc_score(emb2, w1_synt, b1_synt, w2_synt, b2_synt,
              w1_sent, b1_sent, w2_sent, b2_sent, block_b=2048, interpret=False):
    grid = (B // block_b,)
    full = lambda shape: pl.BlockSpec(shape, lambda i: (0, 0))
    row = lambda w: pl.BlockSpec((block_b, w), lambda i: (i, 0))
    out_sd = [jax.ShapeDtypeStruct((B, 1), jnp.float32),
              jax.ShapeDtypeStruct((B, 2), jnp.float32)] * 3
    out_specs = [row(1), row(2)] * 3
    return pl.pallas_call(
        _mlp_body,
        grid=grid,
        in_specs=[
            row(SEQ * D),
            full((3 * D, H)), full((1, H)), full((1, H)), full((1, 1)),
            full((3 * D, H)), full((1, H)), full((2, H)), full((1, 2)),
        ],
        out_specs=out_specs,
        out_shape=out_sd,
        interpret=interpret,
    )(emb2,
      w1_synt, b1_synt.reshape(1, H), w2_synt.T, b2_synt.reshape(1, 1),
      w1_sent, b1_sent.reshape(1, H), w2_sent.T, b2_sent.reshape(1, 2))


def kernel(x, E, w1_synt, b1_synt, w2_synt, b2_synt,
           w1_sent, b1_sent, w2_sent, b2_sent):
    idx_3d = x.reshape(NW, NCHUNK, CHUNK).astype(jnp.int32)
    emb = _sc_gather(idx_3d, E)
    emb2 = emb.reshape(B, SEQ * D)
    outs = _tc_score(emb2, w1_synt, b1_synt, w2_synt, b2_synt,
                     w1_sent, b1_sent, w2_sent, b2_sent)
    return (outs[0], outs[1], outs[2], outs[3], outs[4], outs[5])
